# baseline (speedup 1.0000x reference)
import functools

import jax
import jax.numpy as jnp
from jax import lax
from jax.experimental import pallas as pl
from jax.experimental.pallas import tpu as pltpu
from jax.experimental.pallas import tpu_sc as plsc

NC, NS, L = 2, 16, 16
NW = NC * NS
B, S, D = 4, 2048, 1024
PPT = S // NW
CH = 16
NCH = PPT // CH
NV = D // L
NBUF = 4
STEPS = [(c, b) for c in range(NCH) for b in range(B)]
NIT = len(STEPS)

_mesh = plsc.VectorSubcoreMesh(
    core_axis_name="c", subcore_axis_name="s", num_cores=NC, num_subcores=NS
)


@functools.partial(
    pl.kernel,
    out_type=jax.ShapeDtypeStruct((B, S, D), jnp.float32),
    mesh=_mesh,
    scratch_types=[
        pltpu.VMEM((B, S), jnp.int32),
        pltpu.VMEM((2, CH, D), jnp.float32),
        pltpu.VMEM((NBUF, CH, D), jnp.float32),
        [pltpu.SemaphoreType.DMA] * NBUF,
        [pltpu.SemaphoreType.DMA] * NBUF,
        [pltpu.SemaphoreType.DMA] * 2,
    ],
)
def _emb_kernel(x_hbm, emb_hbm, pos_hbm, out_hbm, idx_v, pos_v, tok_v,
                gs, osems, psems):
    wid = lax.axis_index("s") * NC + lax.axis_index("c")
    pbase = wid * PPT
    gdesc = [None] * NBUF
    odesc = [None] * NBUF
    pdesc = [None, None]

    pltpu.sync_copy(x_hbm, idx_v)

    def start_gather(s):
        c, b = STEPS[s]
        buf = s % NBUF
        gdesc[buf] = pltpu.async_copy(
            emb_hbm.at[idx_v.at[b, pl.ds(pbase + c * CH, CH)]],
            tok_v.at[buf],
            gs[buf],
        )

    def start_pos(c):
        pdesc[c % 2] = pltpu.async_copy(
            pos_hbm.at[pl.ds(pbase + c * CH, CH)],
            pos_v.at[c % 2],
            psems[c % 2],
        )

    def add_pos(buf, pc):
        def row_body(r, carry):
            for j in range(NV):
                sl = pl.ds(j * L, L)
                plsc.addupdate(tok_v.at[buf, r, sl], pos_v[pc, r, sl])
            return carry
        lax.fori_loop(0, CH, row_body, 0)

    start_pos(0)
    for s in range(NBUF - 1):
        start_gather(s)
    for s in range(NIT):
        c, b = STEPS[s]
        buf = s % NBUF
        if b == 0:
            pdesc[c % 2].wait()
            if c + 1 < NCH:
                start_pos(c + 1)
        if s + NBUF - 1 < NIT:
            nbuf = (s + NBUF - 1) % NBUF
            if odesc[nbuf] is not None:
                odesc[nbuf].wait()
            start_gather(s + NBUF - 1)
        gdesc[buf].wait()
        add_pos(buf, c % 2)
        odesc[buf] = pltpu.async_copy(
            tok_v.at[buf],
            out_hbm.at[b, pl.ds(pbase + c * CH, CH)],
            osems[buf],
        )
    for d in odesc:
        d.wait()


def kernel(x, emb, pos_emb):
    return _emb_kernel(jnp.asarray(x, jnp.int32), emb, pos_emb)


# speedup vs baseline: 1.2115x; 1.2115x over previous
import functools

import jax
import jax.numpy as jnp
from jax import lax
from jax.experimental import pallas as pl
from jax.experimental.pallas import tpu as pltpu
from jax.experimental.pallas import tpu_sc as plsc

NC, NS, L = 2, 16, 16
NW = NC * NS
B, S, D = 4, 2048, 1024
PPT = S // NW
CH = 32
NCH = PPT // CH
NV = D // L
STEPS = [(c, b) for c in range(NCH) for b in range(B)]
NIT = len(STEPS)

_mesh = plsc.VectorSubcoreMesh(
    core_axis_name="c", subcore_axis_name="s", num_cores=NC, num_subcores=NS
)


@functools.partial(
    pl.kernel,
    out_type=jax.ShapeDtypeStruct((B, S, D), jnp.float32),
    mesh=_mesh,
    scratch_types=[
        pltpu.VMEM((B, S), jnp.int32),
        pltpu.VMEM((CH, D), jnp.float32),
        pltpu.VMEM((2, CH, D), jnp.float32),
        pltpu.SemaphoreType.DMA,
        pltpu.SemaphoreType.DMA,
        pltpu.SemaphoreType.DMA,
        pltpu.SemaphoreType.DMA,
    ],
)
def _emb_kernel(x_hbm, emb_hbm, pos_hbm, out_hbm, idx_v, pos_v, tok_v,
                g0, g1, o0, o1):
    wid = lax.axis_index("s") * NC + lax.axis_index("c")
    pbase = wid * PPT
    gs = [g0, g1]
    osems = [o0, o1]
    gdesc = [None, None]
    odesc = [None, None]

    pltpu.sync_copy(x_hbm, idx_v)

    def start_gather(s):
        c, b = STEPS[s]
        buf = s % 2
        gdesc[buf] = pltpu.async_copy(
            emb_hbm.at[idx_v.at[b, pl.ds(pbase + c * CH, CH)]],
            tok_v.at[buf],
            gs[buf],
        )

    def add_pos(buf):
        @plsc.parallel_loop(0, CH, 1)
        def _row(r):
            for j in range(NV):
                sl = pl.ds(j * L, L)
                plsc.addupdate(tok_v.at[buf, r, sl], pos_v[r, sl])

    start_gather(0)
    for s in range(NIT):
        c, b = STEPS[s]
        buf = s % 2
        if b == 0:
            pltpu.sync_copy(pos_hbm.at[pl.ds(pbase + c * CH, CH)], pos_v)
        if s + 1 < NIT:
            nbuf = (s + 1) % 2
            if odesc[nbuf] is not None:
                odesc[nbuf].wait()
            start_gather(s + 1)
        gdesc[buf].wait()
        add_pos(buf)
        odesc[buf] = pltpu.async_copy(
            tok_v.at[buf],
            out_hbm.at[b, pl.ds(pbase + c * CH, CH)],
            osems[buf],
        )
    odesc[0].wait()
    odesc[1].wait()


def kernel(x, emb, pos_emb):
    return _emb_kernel(jnp.asarray(x, jnp.int32), emb, pos_emb)


# speedup vs baseline: 1.2847x; 1.0605x over previous
import functools

import jax
import jax.numpy as jnp
from jax import lax
from jax.experimental import pallas as pl
from jax.experimental.pallas import tpu as pltpu
from jax.experimental.pallas import tpu_sc as plsc

NC, NS, L = 2, 16, 16
NW = NC * NS
B, S, D = 4, 2048, 1024
PPT = S // NW
CH = 8
NCH = PPT // CH
NV = D // L

_mesh = plsc.VectorSubcoreMesh(
    core_axis_name="c", subcore_axis_name="s", num_cores=NC, num_subcores=NS
)


@functools.partial(
    pl.kernel,
    out_type=jax.ShapeDtypeStruct((B, S, D), jnp.float32),
    mesh=_mesh,
    scratch_types=[
        pltpu.VMEM((B, S), jnp.int32),
        pltpu.VMEM((2, CH, D), jnp.float32),
        pltpu.VMEM((2, B, CH, D), jnp.float32),
        [pltpu.SemaphoreType.DMA] * 2,
        [pltpu.SemaphoreType.DMA] * 2,
        [pltpu.SemaphoreType.DMA] * 2,
    ],
)
def _emb_kernel(x_hbm, emb_hbm, pos_hbm, out_hbm, idx_v, pos_v, tok_v,
                gs, osems, psems):
    wid = lax.axis_index("s") * NC + lax.axis_index("c")
    pbase = wid * PPT
    gdesc = [[None] * B, [None] * B]
    odesc = [[None] * B, [None] * B]
    pdesc = [None, None]

    pltpu.sync_copy(x_hbm, idx_v)

    def start_quad_gathers(c):
        q = c % 2
        for b in range(B):
            gdesc[q][b] = pltpu.async_copy(
                emb_hbm.at[idx_v.at[b, pl.ds(pbase + c * CH, CH)]],
                tok_v.at[q, b],
                gs[q],
            )

    def start_pos(c):
        pdesc[c % 2] = pltpu.async_copy(
            pos_hbm.at[pl.ds(pbase + c * CH, CH)],
            pos_v.at[c % 2],
            psems[c % 2],
        )

    def quad_add(q):
        def row_body(r, carry):
            for j in range(NV):
                sl = pl.ds(j * L, L)
                pvec = pos_v[q, r, sl]
                for b in range(B):
                    plsc.addupdate(tok_v.at[q, b, r, sl], pvec)
            return carry
        lax.fori_loop(0, CH, row_body, 0)

    start_pos(0)
    start_quad_gathers(0)
    for c in range(NCH):
        q = c % 2
        if c + 1 < NCH:
            nq = (c + 1) % 2
            start_pos(c + 1)
            if odesc[nq][0] is not None:
                for b in range(B):
                    odesc[nq][b].wait()
            start_quad_gathers(c + 1)
        pdesc[q].wait()
        for b in range(B):
            gdesc[q][b].wait()
        quad_add(q)
        for b in range(B):
            odesc[q][b] = pltpu.async_copy(
                tok_v.at[q, b],
                out_hbm.at[b, pl.ds(pbase + c * CH, CH)],
                osems[q],
            )
    for b in range(B):
        odesc[0][b].wait()
        odesc[1][b].wait()


def kernel(x, emb, pos_emb):
    return _emb_kernel(jnp.asarray(x, jnp.int32), emb, pos_emb)


# speedup vs baseline: 1.3030x; 1.0142x over previous
import functools

import jax
import jax.numpy as jnp
from jax import lax
from jax.experimental import pallas as pl
from jax.experimental.pallas import tpu as pltpu
from jax.experimental.pallas import tpu_sc as plsc

NC, NS, L = 2, 16, 16
NW = NC * NS
B, S, D = 4, 2048, 1024
PPT = S // NW
CH = 8
NCH = PPT // CH
NV = D // L
MR = B * CH

_mesh = plsc.VectorSubcoreMesh(
    core_axis_name="c", subcore_axis_name="s", num_cores=NC, num_subcores=NS
)


@functools.partial(
    pl.kernel,
    out_type=jax.ShapeDtypeStruct((B, S, D), jnp.float32),
    mesh=_mesh,
    scratch_types=[
        pltpu.VMEM((NCH, 128), jnp.int32),
        pltpu.VMEM((2, CH, D), jnp.float32),
        pltpu.VMEM((2, MR, D), jnp.float32),
        [pltpu.SemaphoreType.DMA] * 2,
        [pltpu.SemaphoreType.DMA] * 2,
        [pltpu.SemaphoreType.DMA] * 2,
    ],
)
def _emb_kernel(x_hbm, emb_hbm, pos_hbm, out_hbm, ids_v, pos_v, tok_v,
                gs, osems, psems):
    wid = lax.axis_index("s") * NC + lax.axis_index("c")
    pbase = wid * PPT
    gdesc = [None, None]
    odesc = [[None] * B, [None] * B]
    pdesc = [None, None]

    pltpu.sync_copy(x_hbm.at[wid], ids_v)

    def start_gather(c):
        q = c % 2
        gdesc[q] = pltpu.async_copy(
            emb_hbm.at[ids_v.at[c, pl.ds(0, MR)]], tok_v.at[q], gs[q]
        )

    def start_pos(c):
        pdesc[c % 2] = pltpu.async_copy(
            pos_hbm.at[pl.ds(pbase + c * CH, CH)],
            pos_v.at[c % 2],
            psems[c % 2],
        )

    def quad_add(q):
        def row_body(r, carry):
            for j in range(NV):
                sl = pl.ds(j * L, L)
                pvec = pos_v[q, r, sl]
                for b in range(B):
                    plsc.addupdate(tok_v.at[q, b * CH + r, sl], pvec)
            return carry
        lax.fori_loop(0, CH, row_body, 0)

    start_pos(0)
    start_gather(0)
    for c in range(NCH):
        q = c % 2
        if c + 1 < NCH:
            nq = (c + 1) % 2
            start_pos(c + 1)
            if odesc[nq][0] is not None:
                for b in range(B):
                    odesc[nq][b].wait()
            start_gather(c + 1)
        pdesc[q].wait()
        gdesc[q].wait()
        quad_add(q)
        for b in range(B):
            odesc[q][b] = pltpu.async_copy(
                tok_v.at[q, pl.ds(b * CH, CH)],
                out_hbm.at[b, pl.ds(pbase + c * CH, CH)],
                osems[q],
            )
    for b in range(B):
        odesc[0][b].wait()
        odesc[1][b].wait()


def kernel(x, emb, pos_emb):
    xm = jnp.asarray(x, jnp.int32).reshape(B, NW, NCH, CH)
    xm = xm.transpose(1, 2, 0, 3).reshape(NW, NCH, MR)
    xm = jnp.pad(xm, ((0, 0), (0, 0), (0, 128 - MR)))
    return _emb_kernel(xm, emb, pos_emb)


# speedup vs baseline: 1.3055x; 1.0020x over previous
import functools

import jax
import jax.numpy as jnp
from jax import lax
from jax.experimental import pallas as pl
from jax.experimental.pallas import tpu as pltpu
from jax.experimental.pallas import tpu_sc as plsc

NC, NS, L = 2, 16, 16
NW = NC * NS
B, S, D = 4, 2048, 1024
PPT = S // NW
CH = 8
NCH = PPT // CH
NV = D // L
MR = B * CH

_mesh = plsc.VectorSubcoreMesh(
    core_axis_name="c", subcore_axis_name="s", num_cores=NC, num_subcores=NS
)


@functools.partial(
    pl.kernel,
    out_type=jax.ShapeDtypeStruct((B, S, D), jnp.float32),
    mesh=_mesh,
    scratch_types=[
        pltpu.VMEM((NCH, 128), jnp.int32),
        pltpu.VMEM((2, CH, D), jnp.float32),
        pltpu.VMEM((2, MR, D), jnp.float32),
        [pltpu.SemaphoreType.DMA] * 2,
        [pltpu.SemaphoreType.DMA] * 2,
        [pltpu.SemaphoreType.DMA] * 2,
    ],
)
def _emb_kernel(x_hbm, emb_hbm, pos_hbm, out_hbm, ids_v, pos_v, tok_v,
                gs, osems, psems):
    wid = lax.axis_index("s") * NC + lax.axis_index("c")
    pbase = wid * PPT
    gdesc = [None, None]
    odesc = [[None] * B, [None] * B]
    pdesc = [None, None]

    pltpu.sync_copy(x_hbm.at[wid], ids_v)

    def start_gather(c):
        q = c % 2
        gdesc[q] = pltpu.async_copy(
            emb_hbm.at[ids_v.at[c, pl.ds(0, MR)]], tok_v.at[q], gs[q]
        )

    def start_pos(c):
        pdesc[c % 2] = pltpu.async_copy(
            pos_hbm.at[pl.ds(pbase + c * CH, CH)],
            pos_v.at[c % 2],
            psems[c % 2],
        )

    def quad_add(q):
        def row_body(r, carry):
            for j in range(NV):
                sl = pl.ds(j * L, L)
                pvec = pos_v[q, r, sl]
                for b in range(B):
                    plsc.addupdate(tok_v.at[q, b * CH + r, sl], pvec)
            return carry
        lax.fori_loop(0, CH, row_body, 0)

    start_pos(0)
    start_gather(0)
    for c in range(NCH):
        q = c % 2
        if c + 1 < NCH:
            nq = (c + 1) % 2
            start_pos(c + 1)
            if odesc[nq][0] is not None:
                odesc[nq][0].wait()
            start_gather(c + 1)
        pdesc[q].wait()
        gdesc[q].wait()
        quad_add(q)
        odesc[q][0] = pltpu.async_copy(
            tok_v.at[q].reshape(B, CH, D),
            out_hbm.at[:, pl.ds(pbase + c * CH, CH)],
            osems[q],
        )
    odesc[0][0].wait()
    odesc[1][0].wait()


def kernel(x, emb, pos_emb):
    xm = jnp.asarray(x, jnp.int32).reshape(B, NW, NCH, CH)
    xm = xm.transpose(1, 2, 0, 3).reshape(NW, NCH, MR)
    xm = jnp.pad(xm, ((0, 0), (0, 0), (0, 128 - MR)))
    return _emb_kernel(xm, emb, pos_emb)


# speedup vs baseline: 1.3267x; 1.0162x over previous
import functools

import jax
import jax.numpy as jnp
from jax import lax
from jax.experimental import pallas as pl
from jax.experimental.pallas import tpu as pltpu
from jax.experimental.pallas import tpu_sc as plsc

NC, NS, L = 2, 16, 16
NW = NC * NS
B, S, D = 4, 2048, 1024
PPT = S // NW
CH = 8
NCH = PPT // CH
NV = D // L
MR = B * CH

_mesh = plsc.VectorSubcoreMesh(
    core_axis_name="c", subcore_axis_name="s", num_cores=NC, num_subcores=NS
)


@functools.partial(
    pl.kernel,
    out_type=jax.ShapeDtypeStruct((B, S, D), jnp.float32),
    mesh=_mesh,
    scratch_types=[
        pltpu.VMEM((NCH, 128), jnp.int32),
        pltpu.VMEM((2, CH, D), jnp.float32),
        pltpu.VMEM((3, MR, D), jnp.float32),
        [pltpu.SemaphoreType.DMA] * 3,
        [pltpu.SemaphoreType.DMA] * 3,
        [pltpu.SemaphoreType.DMA] * 2,
    ],
)
def _emb_kernel(x_hbm, emb_hbm, pos_hbm, out_hbm, ids_v, pos_v, tok_v,
                gs, osems, psems):
    wid = lax.axis_index("s") * NC + lax.axis_index("c")
    pbase = wid * PPT
    gdesc = [None, None, None]
    odesc = [[None], [None], [None]]
    pdesc = [None, None]

    pltpu.sync_copy(x_hbm.at[wid], ids_v)

    def start_gather(c):
        q = c % 3
        gdesc[q] = pltpu.async_copy(
            emb_hbm.at[ids_v.at[c, pl.ds(0, MR)]], tok_v.at[q], gs[q]
        )

    def start_pos(c):
        pdesc[c % 2] = pltpu.async_copy(
            pos_hbm.at[pl.ds(pbase + c * CH, CH)],
            pos_v.at[c % 2],
            psems[c % 2],
        )

    def quad_add(q, pq):
        def row_body(r, carry):
            for j in range(NV):
                sl = pl.ds(j * L, L)
                pvec = pos_v[pq, r, sl]
                for b in range(B):
                    plsc.addupdate(tok_v.at[q, b * CH + r, sl], pvec)
            return carry
        lax.fori_loop(0, CH, row_body, 0)

    start_pos(0)
    start_gather(0)
    start_gather(1)
    for c in range(NCH):
        q = c % 3
        if c + 1 < NCH:
            start_pos(c + 1)
        if c + 2 < NCH:
            nq = (c + 2) % 3
            if odesc[nq][0] is not None:
                odesc[nq][0].wait()
            start_gather(c + 2)
        pdesc[c % 2].wait()
        gdesc[q].wait()
        quad_add(q, c % 2)
        odesc[q][0] = pltpu.async_copy(
            tok_v.at[q].reshape(B, CH, D),
            out_hbm.at[:, pl.ds(pbase + c * CH, CH)],
            osems[q],
        )
    odesc[0][0].wait()
    odesc[1][0].wait()
    odesc[2][0].wait()


def kernel(x, emb, pos_emb):
    xm = jnp.asarray(x, jnp.int32).reshape(B, NW, NCH, CH)
    xm = xm.transpose(1, 2, 0, 3).reshape(NW, NCH, MR)
    xm = jnp.pad(xm, ((0, 0), (0, 0), (0, 128 - MR)))
    return _emb_kernel(xm, emb, pos_emb)
